# baseline (speedup 1.0000x reference)
import functools

import jax
import jax.numpy as jnp
import numpy as np
from jax import lax
from jax.experimental import pallas as pl
from jax.experimental.pallas import tpu as pltpu
from jax.experimental.pallas import tpu_sc as plsc

N = 50000
E = 800000
F = 32
NSP = 5
NBASIS = 8
SH = 16
RH = 64
RMAX = 4.0
AVG_NEIGH = 16.0


def _cg_consts():
    cg110 = (np.eye(3) / np.sqrt(3.0))[:, :, None]
    eps = np.zeros((3, 3, 3))
    eps[0, 1, 2] = eps[1, 2, 0] = eps[2, 0, 1] = 1.0
    eps[0, 2, 1] = eps[2, 1, 0] = eps[1, 0, 2] = -1.0
    cg111 = eps / np.sqrt(6.0)
    s2 = 1.0 / np.sqrt(2.0)
    s6 = 1.0 / np.sqrt(6.0)
    B = [np.array([[0, s2, 0], [s2, 0, 0], [0, 0, 0]]),
         np.array([[0, 0, 0], [0, 0, s2], [0, s2, 0]]),
         np.array([[-s6, 0, 0], [0, -s6, 0], [0, 0, 2 * s6]]),
         np.array([[0, 0, s2], [0, 0, 0], [s2, 0, 0]]),
         np.array([[s2, 0, 0], [0, -s2, 0], [0, 0, 0]])]
    cg112 = np.stack(B, axis=-1)
    return cg110, cg111, cg112

_CG110, _CG111, _CG112 = _cg_consts()

_DN0 = (((0,), (0,)), ((), ()))


def _swish(x):
    return x * (1.0 / (1.0 + jnp.exp(-x)))


_EB = 4096
_EGRID = 196
_EPAD = _EB * _EGRID


def _gate_body(v_ref, w1_ref, w2_ref, w3_ref, wmix_ref, g_ref):
    v = v_ref[...]
    x = v[0:1, :]
    y = v[1:2, :]
    z = v[2:3, :]
    r2 = x * x + y * y + z * z + 1e-12
    r = jnp.sqrt(r2)
    inv_r = 1.0 / r
    ux = x * inv_r
    uy = y * inv_r
    uz = z * inv_r

    s3 = np.sqrt(3.0); s15 = np.sqrt(15.0); s5 = np.sqrt(5.0)
    c1 = np.sqrt(35.0 / 8.0); c2 = np.sqrt(105.0); c3 = np.sqrt(21.0 / 8.0)
    c4 = np.sqrt(7.0) / 2.0; c5 = np.sqrt(105.0) / 2.0
    xx = ux * ux; yy = uy * uy; zz = uz * uz
    sh = jnp.concatenate([
        jnp.ones_like(ux),
        s3 * ux, s3 * uy, s3 * uz,
        s15 * ux * uy, s15 * uy * uz, (s5 / 2.0) * (3.0 * zz - 1.0),
        s15 * ux * uz, (s15 / 2.0) * (xx - yy),
        c1 * uy * (3.0 * xx - yy), c2 * ux * uy * uz,
        c3 * uy * (5.0 * zz - 1.0), c4 * uz * (5.0 * zz - 3.0),
        c3 * ux * (5.0 * zz - 1.0), c5 * uz * (xx - yy),
        c1 * ux * (xx - 3.0 * yy),
    ], axis=0)

    u = r * (1.0 / RMAX)
    scale = np.sqrt(2.0 / RMAX) * inv_r
    sins = jnp.concatenate(
        [jnp.sin((float(n) * np.pi) * u) for n in range(1, NBASIS + 1)],
        axis=0)
    u2 = u * u; u3 = u2 * u; u6 = u3 * u3; u7 = u6 * u; u8 = u7 * u
    env = 1.0 - 28.0 * u6 + 48.0 * u7 - 21.0 * u8
    env = jnp.where(u < 1.0, env, 0.0)
    radial = sins * (scale * env)

    rw = lax.dot_general(w1_ref[...], radial, _DN0,
                         preferred_element_type=jnp.float32)
    rw = _swish(rw)
    rw = lax.dot_general(w2_ref[...], rw, _DN0,
                         preferred_element_type=jnp.float32)
    rw = _swish(rw)
    rw = lax.dot_general(w3_ref[...], rw, _DN0,
                         preferred_element_type=jnp.float32)
    coeff = rw * sh
    coeff4 = coeff.reshape(SH * 4, _EB // 4)
    g4 = lax.dot_general(coeff4, wmix_ref[...], _DN0,
                         preferred_element_type=jnp.float32)
    g_ref[...] = g4.reshape(_EB * F)


def _k_gate(vectors_t, w1, w2, w3, wmix):
    full = lambda shape: pl.BlockSpec(shape, lambda i: (0,) * len(shape))
    return pl.pallas_call(
        _gate_body,
        grid=(_EGRID,),
        in_specs=[
            pl.BlockSpec((3, _EB), lambda i: (0, i)),
            full((NBASIS, RH)), full((RH, RH)), full((RH, SH)),
            full((SH * 4, F * 4)),
        ],
        out_specs=pl.BlockSpec((_EB * F,), lambda i: (i,)),
        out_shape=jax.ShapeDtypeStruct((_EPAD * F,), jnp.float32),
    )(vectors_t, w1, w2, w3, wmix)


_NB = 2000
_NGRID = N // _NB


def _h0_body(sp_ref, table_ref, out_ref):
    sp = sp_ref[0]
    iot = lax.broadcasted_iota(jnp.int32, (NSP, _NB), 0)
    oh = (iot == sp).astype(jnp.float32)
    out_ref[...] = lax.dot_general(oh, table_ref[...], _DN0,
                                   preferred_element_type=jnp.float32)


def _k_h0(specie_r, table):
    return pl.pallas_call(
        _h0_body,
        grid=(_NGRID,),
        in_specs=[pl.BlockSpec((1, 1, _NB), lambda i: (i, 0, 0)),
                  pl.BlockSpec((NSP, F), lambda i: (0, 0))],
        out_specs=pl.BlockSpec((_NB, F), lambda i: (i, 0)),
        out_shape=jax.ShapeDtypeStruct((N, F), jnp.float32),
    )(specie_r, table)


_NCORES = 2
_NSUB = 16
_NWORK = _NCORES * _NSUB
_EPW = E // _NWORK
_CH = 128
_NFULL = _EPW // _CH
_TAIL = _EPW - _NFULL * _CH
_NPAD = 51200
_RPS = _NPAD // _NSUB


def _msg_body(h_hbm, gate_hbm, snd_hbm, rcv_hbm, out_hbm,
              s0, s1, r0, r1, r2, r3, g0v, g1v, h0v, h1v, m0, m1,
              sidx_t, ridx_t,
              agg_sh, sem_in0, sem_in1, sem_g0, sem_g1, sem_s0, sem_s1,
              sem_z):
    c = lax.axis_index("c")
    s = lax.axis_index("s")
    w = c * _NSUB + s
    base_w = w * _EPW

    svs = (s0, s1)
    rvs = (r0, r1, r2, r3)
    gvs = (g0v, g1v)
    hvs = (h0v, h1v)
    mvs = (m0, m1)
    sem_ins = (sem_in0, sem_in1)
    sem_gs = (sem_g0, sem_g1)
    sem_ss = (sem_s0, sem_s1)

    def issue_inputs(cidx, k2, k4):
        base = base_w + cidx * _CH
        pltpu.async_copy(snd_hbm.at[pl.ds(base, _CH)], svs[k2], sem_ins[k2])
        pltpu.async_copy(rcv_hbm.at[pl.ds(base, _CH)], rvs[k4], sem_ins[k2])
        pltpu.async_copy(gate_hbm.at[pl.ds(base * F, _CH * F)], gvs[k2],
                         sem_ins[k2])

    def wait_inputs(k2):
        pltpu.make_async_copy(snd_hbm.at[pl.ds(0, _CH)], svs[k2],
                              sem_ins[k2]).wait()
        pltpu.make_async_copy(rcv_hbm.at[pl.ds(0, _CH)], rvs[0],
                              sem_ins[k2]).wait()
        pltpu.make_async_copy(gate_hbm.at[pl.ds(0, _CH * F)], gvs[k2],
                              sem_ins[k2]).wait()

    def issue_gather(k2):
        pltpu.async_copy(h_hbm.at[svs[k2]], hvs[k2], sem_gs[k2])

    def drain_scatter(k2, k4):
        pltpu.make_async_copy(mvs[k2], agg_sh.at[rvs[k4]], sem_ss[k2]).wait()

    issue_inputs(0, 0, 0)
    issue_inputs(1, 1, 1)

    @plsc.parallel_loop(0, _CH, unroll=8)
    def _zrow(i):
        m0[i, pl.ds(0, 16)] = jnp.zeros((16,), jnp.float32)
        m0[i, pl.ds(16, 16)] = jnp.zeros((16,), jnp.float32)

    for t in range(_RPS // _CH):
        pltpu.async_copy(m0, agg_sh.at[pl.ds(s * _RPS + t * _CH, _CH)], sem_z)
    for t in range(_RPS // _CH):
        pltpu.make_async_copy(m0, agg_sh.at[pl.ds(s * _RPS, _CH)], sem_z).wait()
    plsc.subcore_barrier()

    wait_inputs(0)
    issue_gather(0)

    def process(cidx, k2, k4, drain, prefetch, nxt):
        if drain:
            drain_scatter(k2, (k4 + 2) % 4)
        pltpu.make_async_copy(h_hbm.at[svs[k2]], hvs[k2], sem_gs[k2]).wait()

        hv, gv, mv = hvs[k2], gvs[k2], mvs[k2]

        @plsc.parallel_loop(0, _CH, unroll=8)
        def _mul(i):
            mv[i, pl.ds(0, 16)] = hv[i, pl.ds(0, 16)] * gv[pl.ds(i * F, 16)]
            mv[i, pl.ds(16, 16)] = (hv[i, pl.ds(16, 16)]
                                    * gv[pl.ds(i * F + 16, 16)])

        pltpu.async_copy(mv, agg_sh.at[rvs[k4]], sem_ss[k2], add=True)
        if prefetch:
            issue_inputs(cidx + 2, k2, (k4 + 2) % 4)
        if nxt:
            wait_inputs(1 - k2)
            issue_gather(1 - k2)

    _NQ = _NFULL // 4
    _REM = _NFULL - _NQ * 4

    for k in range(4):
        process(k, k % 2, k, drain=k >= 2, prefetch=True, nxt=True)

    def _quad(q, _):
        cbase = q * 4
        for k in range(4):
            process(cbase + k, k % 2, k, drain=True, prefetch=True, nxt=True)
        return 0
    lax.fori_loop(1, _NQ, _quad, 0)

    for k in range(_REM):
        cidx = _NQ * 4 + k
        process(cidx, k % 2, k, drain=True,
                prefetch=cidx + 2 < _NFULL, nxt=cidx + 1 < _NFULL)

    drain_scatter((_NFULL - 2) % 2, (_NFULL - 2) % 4)
    drain_scatter((_NFULL - 1) % 2, (_NFULL - 1) % 4)

    tbase = base_w + _NFULL * _CH
    pltpu.sync_copy(snd_hbm.at[pl.ds(tbase, _TAIL)], sidx_t)
    pltpu.sync_copy(rcv_hbm.at[pl.ds(tbase, _TAIL)], ridx_t)
    pltpu.async_copy(h_hbm.at[sidx_t], h0v.at[pl.ds(0, _TAIL)], sem_g0).wait()
    pltpu.sync_copy(gate_hbm.at[pl.ds(tbase * F, _TAIL * F)],
                    g0v.at[pl.ds(0, _TAIL * F)])

    @plsc.parallel_loop(0, _TAIL, unroll=8)
    def _mul_t(i):
        m0[i, pl.ds(0, 16)] = h0v[i, pl.ds(0, 16)] * g0v[pl.ds(i * F, 16)]
        m0[i, pl.ds(16, 16)] = (h0v[i, pl.ds(16, 16)]
                                * g0v[pl.ds(i * F + 16, 16)])

    pltpu.sync_copy(m0.at[pl.ds(0, _TAIL)], agg_sh.at[ridx_t], add=True)

    plsc.subcore_barrier()
    pltpu.sync_copy(agg_sh.at[pl.ds(s * _RPS, _RPS)],
                    out_hbm.at[pl.ds(c * _NPAD + s * _RPS, _RPS)])


@functools.lru_cache(maxsize=None)
def _msg_call():
    return pl.kernel(
        _msg_body,
        mesh=plsc.VectorSubcoreMesh(core_axis_name="c", subcore_axis_name="s"),
        compiler_params=pltpu.CompilerParams(use_tc_tiling_on_sc=False),
        out_type=jax.ShapeDtypeStruct((_NCORES * _NPAD, F), jnp.float32),
        scratch_types=[
            pltpu.VMEM((_CH,), jnp.int32),
            pltpu.VMEM((_CH,), jnp.int32),
            pltpu.VMEM((_CH,), jnp.int32),
            pltpu.VMEM((_CH,), jnp.int32),
            pltpu.VMEM((_CH,), jnp.int32),
            pltpu.VMEM((_CH,), jnp.int32),
            pltpu.VMEM((_CH * F,), jnp.float32),
            pltpu.VMEM((_CH * F,), jnp.float32),
            pltpu.VMEM((_CH, F), jnp.float32),
            pltpu.VMEM((_CH, F), jnp.float32),
            pltpu.VMEM((_CH, F), jnp.float32),
            pltpu.VMEM((_CH, F), jnp.float32),
            pltpu.VMEM((_TAIL,), jnp.int32),
            pltpu.VMEM((_TAIL,), jnp.int32),
            pltpu.VMEM_SHARED((_NPAD, F), jnp.float32),
            pltpu.SemaphoreType.DMA,
            pltpu.SemaphoreType.DMA,
            pltpu.SemaphoreType.DMA,
            pltpu.SemaphoreType.DMA,
            pltpu.SemaphoreType.DMA,
            pltpu.SemaphoreType.DMA,
            pltpu.SemaphoreType.DMA,
        ])


def _k_msg(h, gate, snd, rcv):
    return _msg_call()(h, gate, snd, rcv)


def _node_body(p0_ref, p1_ref, h_ref, wout_ref, wsc_ref, out_ref):
    agg = (p0_ref[...] + p1_ref[...]) * (1.0 / AVG_NEIGH)
    out_ref[...] = _swish(
        jnp.dot(agg, wout_ref[...], preferred_element_type=jnp.float32)
        + jnp.dot(h_ref[...], wsc_ref[...], preferred_element_type=jnp.float32))


_NBN = 2048
_GN = _NPAD // _NBN


def _k_node(p, h, wout, wsc):
    full = lambda shape: pl.BlockSpec(shape, lambda i: (0,) * len(shape))
    return pl.pallas_call(
        _node_body,
        grid=(_GN,),
        in_specs=[pl.BlockSpec((_NBN, F), lambda i: (i, 0)),
                  pl.BlockSpec((_NBN, F), lambda i: (i + _GN, 0)),
                  pl.BlockSpec((_NBN, F), lambda i: (i, 0)),
                  full((F, F)), full((F, F))],
        out_specs=pl.BlockSpec((_NBN, F), lambda i: (i, 0)),
        out_shape=jax.ShapeDtypeStruct((N, F), jnp.float32),
    )(p, p, h, wout, wsc)


def _final_body(p0_ref, p1_ref, h_ref, wout_ref, wsc_ref,
                wb_ref, we_ref, wd_ref, becs_ref, eps_ref, den_ref):
    agg = (p0_ref[...] + p1_ref[...]) * (1.0 / AVG_NEIGH)
    h2 = _swish(
        jnp.dot(agg, wout_ref[...], preferred_element_type=jnp.float32)
        + jnp.dot(h_ref[...], wsc_ref[...], preferred_element_type=jnp.float32))
    becs_ref[...] = jnp.dot(h2, wb_ref[...], preferred_element_type=jnp.float32)
    eps_ref[...] = jnp.dot(h2, we_ref[...], preferred_element_type=jnp.float32)
    den_ref[...] = jnp.dot(h2, wd_ref[...], preferred_element_type=jnp.float32)


def _k_final(p, h, wout, wsc, wb, we, wd):
    full = lambda shape: pl.BlockSpec(shape, lambda i: (0,) * len(shape))
    return pl.pallas_call(
        _final_body,
        grid=(_GN,),
        in_specs=[pl.BlockSpec((_NBN, F), lambda i: (i, 0)),
                  pl.BlockSpec((_NBN, F), lambda i: (i + _GN, 0)),
                  pl.BlockSpec((_NBN, F), lambda i: (i, 0)),
                  full((F, F)), full((F, F)),
                  full((F, 9)), full((F, 9)), full((F, 3))],
        out_specs=[pl.BlockSpec((_NBN, 9), lambda i: (i, 0)),
                   pl.BlockSpec((_NBN, 9), lambda i: (i, 0)),
                   pl.BlockSpec((_NBN, 3), lambda i: (i, 0))],
        out_shape=[jax.ShapeDtypeStruct((N, 9), jnp.float32),
                   jax.ShapeDtypeStruct((N, 9), jnp.float32),
                   jax.ShapeDtypeStruct((N, 3), jnp.float32)],
    )(p, p, h, wout, wsc, wb, we, wd)


def kernel(vectors, node_specie, senders, receivers, W_embed, W_lin,
           R_W1, R_W2, R_W3, W_mix, W_out, W_sc,
           W_b0, W_b1, W_b2, W_e0, W_e1, W_e2, W_den):
    senders = senders.astype(jnp.int32)
    receivers = receivers.astype(jnp.int32)
    vpad = jnp.pad(vectors, ((0, _EPAD - E), (0, 0)))
    v4 = vpad.reshape(_EGRID, _EB // 4, 4, 3)
    vectors_t = jnp.transpose(v4, (3, 0, 2, 1)).reshape(3, _EPAD)
    specie_r = node_specie.astype(jnp.int32).reshape(_NGRID, 1, _NB)

    cg110 = jnp.asarray(_CG110.reshape(9, 1), jnp.float32)
    cg111 = jnp.asarray(_CG111.reshape(9, 3), jnp.float32)
    cg112 = jnp.asarray(_CG112.reshape(9, 5), jnp.float32)
    w_becs = W_b0 @ cg110.T + W_b1 @ cg111.T + W_b2 @ cg112.T
    w_eps = W_e0 @ cg110.T + W_e1 @ cg111.T + W_e2 @ cg112.T
    table = W_embed @ W_lin

    eye4 = jnp.eye(4, dtype=jnp.float32)
    w2 = jnp.einsum("skf,jJ->skjJf", W_mix, eye4).reshape(2, SH * 4, 4 * F)

    gate0 = _k_gate(vectors_t, R_W1[0], R_W2[0], R_W3[0], w2[0])
    h = _k_h0(specie_r, table)
    p = _k_msg(h, gate0, senders, receivers)
    gate1 = _k_gate(vectors_t, R_W1[1], R_W2[1], R_W3[1], w2[1])
    h = _k_node(p, h, W_out[0], W_sc[0])
    p = _k_msg(h, gate1, senders, receivers)
    becs9, eps9, den = _k_final(p, h, W_out[1], W_sc[1], w_becs, w_eps, W_den)
    return becs9.reshape(N, 3, 3), eps9.reshape(N, 3, 3), den


# speedup vs baseline: 1.0950x; 1.0950x over previous
import functools

import jax
import jax.numpy as jnp
import numpy as np
from jax import lax
from jax.experimental import pallas as pl
from jax.experimental.pallas import tpu as pltpu
from jax.experimental.pallas import tpu_sc as plsc

N = 50000
E = 800000
F = 32
NSP = 5
NBASIS = 8
SH = 16
RH = 64
RMAX = 4.0
AVG_NEIGH = 16.0


def _cg_consts():
    cg110 = (np.eye(3) / np.sqrt(3.0))[:, :, None]
    eps = np.zeros((3, 3, 3))
    eps[0, 1, 2] = eps[1, 2, 0] = eps[2, 0, 1] = 1.0
    eps[0, 2, 1] = eps[2, 1, 0] = eps[1, 0, 2] = -1.0
    cg111 = eps / np.sqrt(6.0)
    s2 = 1.0 / np.sqrt(2.0)
    s6 = 1.0 / np.sqrt(6.0)
    B = [np.array([[0, s2, 0], [s2, 0, 0], [0, 0, 0]]),
         np.array([[0, 0, 0], [0, 0, s2], [0, s2, 0]]),
         np.array([[-s6, 0, 0], [0, -s6, 0], [0, 0, 2 * s6]]),
         np.array([[0, 0, s2], [0, 0, 0], [s2, 0, 0]]),
         np.array([[s2, 0, 0], [0, -s2, 0], [0, 0, 0]])]
    cg112 = np.stack(B, axis=-1)
    return cg110, cg111, cg112

_CG110, _CG111, _CG112 = _cg_consts()

_DN0 = (((0,), (0,)), ((), ()))


def _swish(x):
    return x * (1.0 / (1.0 + jnp.exp(-x)))


_EB = 4096
_EGRID = 196
_EPAD = _EB * _EGRID


def _gate_body(v_ref, w1_ref, w2_ref, w3_ref, wmix_ref, g0_ref, g1_ref):
    v = v_ref[...]
    x = v[0:1, :]
    y = v[1:2, :]
    z = v[2:3, :]
    r2 = x * x + y * y + z * z + 1e-12
    r = jnp.sqrt(r2)
    inv_r = 1.0 / r
    ux = x * inv_r
    uy = y * inv_r
    uz = z * inv_r

    s3 = np.sqrt(3.0); s15 = np.sqrt(15.0); s5 = np.sqrt(5.0)
    c1 = np.sqrt(35.0 / 8.0); c2 = np.sqrt(105.0); c3 = np.sqrt(21.0 / 8.0)
    c4 = np.sqrt(7.0) / 2.0; c5 = np.sqrt(105.0) / 2.0
    xx = ux * ux; yy = uy * uy; zz = uz * uz
    sh = jnp.concatenate([
        jnp.ones_like(ux),
        s3 * ux, s3 * uy, s3 * uz,
        s15 * ux * uy, s15 * uy * uz, (s5 / 2.0) * (3.0 * zz - 1.0),
        s15 * ux * uz, (s15 / 2.0) * (xx - yy),
        c1 * uy * (3.0 * xx - yy), c2 * ux * uy * uz,
        c3 * uy * (5.0 * zz - 1.0), c4 * uz * (5.0 * zz - 3.0),
        c3 * ux * (5.0 * zz - 1.0), c5 * uz * (xx - yy),
        c1 * ux * (xx - 3.0 * yy),
    ], axis=0)

    u = r * (1.0 / RMAX)
    scale = np.sqrt(2.0 / RMAX) * inv_r
    sins = jnp.concatenate(
        [jnp.sin((float(n) * np.pi) * u) for n in range(1, NBASIS + 1)],
        axis=0)
    u2 = u * u; u3 = u2 * u; u6 = u3 * u3; u7 = u6 * u; u8 = u7 * u
    env = 1.0 - 28.0 * u6 + 48.0 * u7 - 21.0 * u8
    env = jnp.where(u < 1.0, env, 0.0)
    radial = sins * (scale * env)

    for s, out in ((0, g0_ref), (1, g1_ref)):
        rw = lax.dot_general(w1_ref[s], radial, _DN0,
                             preferred_element_type=jnp.float32)
        rw = _swish(rw)
        rw = lax.dot_general(w2_ref[s], rw, _DN0,
                             preferred_element_type=jnp.float32)
        rw = _swish(rw)
        rw = lax.dot_general(w3_ref[s], rw, _DN0,
                             preferred_element_type=jnp.float32)
        coeff = rw * sh
        coeff4 = coeff.reshape(SH * 4, _EB // 4)
        g4 = lax.dot_general(coeff4, wmix_ref[s], _DN0,
                             preferred_element_type=jnp.float32)
        out[...] = g4.reshape(_EB * F)


def _k_gate(vectors_t, R_W1, R_W2, R_W3, W_mix):
    full = lambda shape: pl.BlockSpec(shape, lambda i: (0,) * len(shape))
    return pl.pallas_call(
        _gate_body,
        grid=(_EGRID,),
        in_specs=[
            pl.BlockSpec((3, _EB), lambda i: (0, i)),
            full((2, NBASIS, RH)), full((2, RH, RH)), full((2, RH, SH)),
            full((2, SH * 4, F * 4)),
        ],
        out_specs=[pl.BlockSpec((_EB * F,), lambda i: (i,)),
                   pl.BlockSpec((_EB * F,), lambda i: (i,))],
        out_shape=[jax.ShapeDtypeStruct((_EPAD * F,), jnp.float32),
                   jax.ShapeDtypeStruct((_EPAD * F,), jnp.float32)],
    )(vectors_t, R_W1, R_W2, R_W3, W_mix)


_NB = 2000
_NGRID = N // _NB


def _h0_body(sp_ref, table_ref, out_ref):
    sp = sp_ref[0]
    iot = lax.broadcasted_iota(jnp.int32, (NSP, _NB), 0)
    oh = (iot == sp).astype(jnp.float32)
    out_ref[...] = lax.dot_general(oh, table_ref[...], _DN0,
                                   preferred_element_type=jnp.float32)


def _k_h0(specie_r, table):
    return pl.pallas_call(
        _h0_body,
        grid=(_NGRID,),
        in_specs=[pl.BlockSpec((1, 1, _NB), lambda i: (i, 0, 0)),
                  pl.BlockSpec((NSP, F), lambda i: (0, 0))],
        out_specs=pl.BlockSpec((_NB, F), lambda i: (i, 0)),
        out_shape=jax.ShapeDtypeStruct((N, F), jnp.float32),
    )(specie_r, table)


_NCORES = 2
_NSUB = 16
_NWORK = _NCORES * _NSUB
_EPW = E // _NWORK
_CH = 128
_NFULL = _EPW // _CH
_TAIL = _EPW - _NFULL * _CH
_NPAD = 51200
_RPS = _NPAD // _NSUB


def _msg_body(h_hbm, gate_hbm, snd_hbm, rcv_hbm, out_hbm,
              s0, s1, r0, r1, r2, r3, g0v, g1v, h0v, h1v, m0, m1,
              sidx_t, ridx_t,
              agg_sh, sem_in0, sem_in1, sem_g0, sem_g1, sem_s0, sem_s1,
              sem_z):
    c = lax.axis_index("c")
    s = lax.axis_index("s")
    w = c * _NSUB + s
    base_w = w * _EPW

    svs = (s0, s1)
    rvs = (r0, r1, r2, r3)
    gvs = (g0v, g1v)
    hvs = (h0v, h1v)
    mvs = (m0, m1)
    sem_ins = (sem_in0, sem_in1)
    sem_gs = (sem_g0, sem_g1)
    sem_ss = (sem_s0, sem_s1)

    def issue_inputs(cidx, k2, k4):
        base = base_w + cidx * _CH
        pltpu.async_copy(snd_hbm.at[pl.ds(base, _CH)], svs[k2], sem_ins[k2])
        pltpu.async_copy(rcv_hbm.at[pl.ds(base, _CH)], rvs[k4], sem_ins[k2])
        pltpu.async_copy(gate_hbm.at[pl.ds(base * F, _CH * F)], gvs[k2],
                         sem_ins[k2])

    def wait_inputs(k2):
        pltpu.make_async_copy(snd_hbm.at[pl.ds(0, _CH)], svs[k2],
                              sem_ins[k2]).wait()
        pltpu.make_async_copy(rcv_hbm.at[pl.ds(0, _CH)], rvs[0],
                              sem_ins[k2]).wait()
        pltpu.make_async_copy(gate_hbm.at[pl.ds(0, _CH * F)], gvs[k2],
                              sem_ins[k2]).wait()

    def issue_gather(k2):
        pltpu.async_copy(h_hbm.at[svs[k2]], hvs[k2], sem_gs[k2])

    def drain_scatter(k2, k4):
        pltpu.make_async_copy(mvs[k2], agg_sh.at[rvs[k4]], sem_ss[k2]).wait()

    issue_inputs(0, 0, 0)
    issue_inputs(1, 1, 1)

    @plsc.parallel_loop(0, _CH, unroll=8)
    def _zrow(i):
        m0[i, pl.ds(0, 16)] = jnp.zeros((16,), jnp.float32)
        m0[i, pl.ds(16, 16)] = jnp.zeros((16,), jnp.float32)

    for t in range(_RPS // _CH):
        pltpu.async_copy(m0, agg_sh.at[pl.ds(s * _RPS + t * _CH, _CH)], sem_z)
    for t in range(_RPS // _CH):
        pltpu.make_async_copy(m0, agg_sh.at[pl.ds(s * _RPS, _CH)], sem_z).wait()
    plsc.subcore_barrier()

    wait_inputs(0)
    issue_gather(0)

    def process(cidx, k2, k4, drain, prefetch, nxt):
        if drain:
            drain_scatter(k2, (k4 + 2) % 4)
        pltpu.make_async_copy(h_hbm.at[svs[k2]], hvs[k2], sem_gs[k2]).wait()

        hv, gv, mv = hvs[k2], gvs[k2], mvs[k2]

        @plsc.parallel_loop(0, _CH, unroll=8)
        def _mul(i):
            mv[i, pl.ds(0, 16)] = hv[i, pl.ds(0, 16)] * gv[pl.ds(i * F, 16)]
            mv[i, pl.ds(16, 16)] = (hv[i, pl.ds(16, 16)]
                                    * gv[pl.ds(i * F + 16, 16)])

        pltpu.async_copy(mv, agg_sh.at[rvs[k4]], sem_ss[k2], add=True)
        if prefetch:
            issue_inputs(cidx + 2, k2, (k4 + 2) % 4)
        if nxt:
            wait_inputs(1 - k2)
            issue_gather(1 - k2)

    _NQ = _NFULL // 4
    _REM = _NFULL - _NQ * 4

    for k in range(4):
        process(k, k % 2, k, drain=k >= 2, prefetch=True, nxt=True)

    def _quad(q, _):
        cbase = q * 4
        for k in range(4):
            process(cbase + k, k % 2, k, drain=True, prefetch=True, nxt=True)
        return 0
    lax.fori_loop(1, _NQ, _quad, 0)

    for k in range(_REM):
        cidx = _NQ * 4 + k
        process(cidx, k % 2, k, drain=True,
                prefetch=cidx + 2 < _NFULL, nxt=cidx + 1 < _NFULL)

    drain_scatter((_NFULL - 2) % 2, (_NFULL - 2) % 4)
    drain_scatter((_NFULL - 1) % 2, (_NFULL - 1) % 4)

    tbase = base_w + _NFULL * _CH
    pltpu.sync_copy(snd_hbm.at[pl.ds(tbase, _TAIL)], sidx_t)
    pltpu.sync_copy(rcv_hbm.at[pl.ds(tbase, _TAIL)], ridx_t)
    pltpu.async_copy(h_hbm.at[sidx_t], h0v.at[pl.ds(0, _TAIL)], sem_g0).wait()
    pltpu.sync_copy(gate_hbm.at[pl.ds(tbase * F, _TAIL * F)],
                    g0v.at[pl.ds(0, _TAIL * F)])

    @plsc.parallel_loop(0, _TAIL, unroll=8)
    def _mul_t(i):
        m0[i, pl.ds(0, 16)] = h0v[i, pl.ds(0, 16)] * g0v[pl.ds(i * F, 16)]
        m0[i, pl.ds(16, 16)] = (h0v[i, pl.ds(16, 16)]
                                * g0v[pl.ds(i * F + 16, 16)])

    pltpu.sync_copy(m0.at[pl.ds(0, _TAIL)], agg_sh.at[ridx_t], add=True)

    plsc.subcore_barrier()
    pltpu.sync_copy(agg_sh.at[pl.ds(s * _RPS, _RPS)],
                    out_hbm.at[pl.ds(c * _NPAD + s * _RPS, _RPS)])


@functools.lru_cache(maxsize=None)
def _msg_call():
    return pl.kernel(
        _msg_body,
        mesh=plsc.VectorSubcoreMesh(core_axis_name="c", subcore_axis_name="s"),
        compiler_params=pltpu.CompilerParams(use_tc_tiling_on_sc=False),
        out_type=jax.ShapeDtypeStruct((_NCORES * _NPAD, F), jnp.float32),
        scratch_types=[
            pltpu.VMEM((_CH,), jnp.int32),
            pltpu.VMEM((_CH,), jnp.int32),
            pltpu.VMEM((_CH,), jnp.int32),
            pltpu.VMEM((_CH,), jnp.int32),
            pltpu.VMEM((_CH,), jnp.int32),
            pltpu.VMEM((_CH,), jnp.int32),
            pltpu.VMEM((_CH * F,), jnp.float32),
            pltpu.VMEM((_CH * F,), jnp.float32),
            pltpu.VMEM((_CH, F), jnp.float32),
            pltpu.VMEM((_CH, F), jnp.float32),
            pltpu.VMEM((_CH, F), jnp.float32),
            pltpu.VMEM((_CH, F), jnp.float32),
            pltpu.VMEM((_TAIL,), jnp.int32),
            pltpu.VMEM((_TAIL,), jnp.int32),
            pltpu.VMEM_SHARED((_NPAD, F), jnp.float32),
            pltpu.SemaphoreType.DMA,
            pltpu.SemaphoreType.DMA,
            pltpu.SemaphoreType.DMA,
            pltpu.SemaphoreType.DMA,
            pltpu.SemaphoreType.DMA,
            pltpu.SemaphoreType.DMA,
            pltpu.SemaphoreType.DMA,
        ])


def _k_msg(h, gate, snd, rcv):
    return _msg_call()(h, gate, snd, rcv)


def _node_body(p0_ref, p1_ref, h_ref, wout_ref, wsc_ref, out_ref):
    agg = (p0_ref[...] + p1_ref[...]) * (1.0 / AVG_NEIGH)
    out_ref[...] = _swish(
        jnp.dot(agg, wout_ref[...], preferred_element_type=jnp.float32)
        + jnp.dot(h_ref[...], wsc_ref[...], preferred_element_type=jnp.float32))


_NBN = 2048
_GN = _NPAD // _NBN


def _k_node(p, h, wout, wsc):
    full = lambda shape: pl.BlockSpec(shape, lambda i: (0,) * len(shape))
    return pl.pallas_call(
        _node_body,
        grid=(_GN,),
        in_specs=[pl.BlockSpec((_NBN, F), lambda i: (i, 0)),
                  pl.BlockSpec((_NBN, F), lambda i: (i + _GN, 0)),
                  pl.BlockSpec((_NBN, F), lambda i: (i, 0)),
                  full((F, F)), full((F, F))],
        out_specs=pl.BlockSpec((_NBN, F), lambda i: (i, 0)),
        out_shape=jax.ShapeDtypeStruct((N, F), jnp.float32),
    )(p, p, h, wout, wsc)


def _final_body(p0_ref, p1_ref, h_ref, wout_ref, wsc_ref,
                wb_ref, we_ref, wd_ref, becs_ref, eps_ref, den_ref):
    agg = (p0_ref[...] + p1_ref[...]) * (1.0 / AVG_NEIGH)
    h2 = _swish(
        jnp.dot(agg, wout_ref[...], preferred_element_type=jnp.float32)
        + jnp.dot(h_ref[...], wsc_ref[...], preferred_element_type=jnp.float32))
    becs_ref[...] = jnp.dot(h2, wb_ref[...], preferred_element_type=jnp.float32)
    eps_ref[...] = jnp.dot(h2, we_ref[...], preferred_element_type=jnp.float32)
    den_ref[...] = jnp.dot(h2, wd_ref[...], preferred_element_type=jnp.float32)


def _k_final(p, h, wout, wsc, wb, we, wd):
    full = lambda shape: pl.BlockSpec(shape, lambda i: (0,) * len(shape))
    return pl.pallas_call(
        _final_body,
        grid=(_GN,),
        in_specs=[pl.BlockSpec((_NBN, F), lambda i: (i, 0)),
                  pl.BlockSpec((_NBN, F), lambda i: (i + _GN, 0)),
                  pl.BlockSpec((_NBN, F), lambda i: (i, 0)),
                  full((F, F)), full((F, F)),
                  full((F, 9)), full((F, 9)), full((F, 3))],
        out_specs=[pl.BlockSpec((_NBN, 9), lambda i: (i, 0)),
                   pl.BlockSpec((_NBN, 9), lambda i: (i, 0)),
                   pl.BlockSpec((_NBN, 3), lambda i: (i, 0))],
        out_shape=[jax.ShapeDtypeStruct((N, 9), jnp.float32),
                   jax.ShapeDtypeStruct((N, 9), jnp.float32),
                   jax.ShapeDtypeStruct((N, 3), jnp.float32)],
    )(p, p, h, wout, wsc, wb, we, wd)


def kernel(vectors, node_specie, senders, receivers, W_embed, W_lin,
           R_W1, R_W2, R_W3, W_mix, W_out, W_sc,
           W_b0, W_b1, W_b2, W_e0, W_e1, W_e2, W_den):
    senders = senders.astype(jnp.int32)
    receivers = receivers.astype(jnp.int32)
    vpad = jnp.pad(vectors, ((0, _EPAD - E), (0, 0)))
    v4 = vpad.reshape(_EGRID, _EB // 4, 4, 3)
    vectors_t = jnp.transpose(v4, (3, 0, 2, 1)).reshape(3, _EPAD)
    specie_r = node_specie.astype(jnp.int32).reshape(_NGRID, 1, _NB)

    cg110 = jnp.asarray(_CG110.reshape(9, 1), jnp.float32)
    cg111 = jnp.asarray(_CG111.reshape(9, 3), jnp.float32)
    cg112 = jnp.asarray(_CG112.reshape(9, 5), jnp.float32)
    w_becs = W_b0 @ cg110.T + W_b1 @ cg111.T + W_b2 @ cg112.T
    w_eps = W_e0 @ cg110.T + W_e1 @ cg111.T + W_e2 @ cg112.T
    table = W_embed @ W_lin

    eye4 = jnp.eye(4, dtype=jnp.float32)
    w2 = jnp.einsum("skf,jJ->skjJf", W_mix, eye4).reshape(2, SH * 4, 4 * F)

    gate0, gate1 = _k_gate(vectors_t, R_W1, R_W2, R_W3, w2)
    h = _k_h0(specie_r, table)
    p = _k_msg(h, gate0, senders, receivers)
    h = _k_node(p, h, W_out[0], W_sc[0])
    p = _k_msg(h, gate1, senders, receivers)
    becs9, eps9, den = _k_final(p, h, W_out[1], W_sc[1], w_becs, w_eps, W_den)
    return becs9.reshape(N, 3, 3), eps9.reshape(N, 3, 3), den
